# Initial kernel scaffold; baseline (speedup 1.0000x reference)
import jax
import jax.numpy as jnp
from jax.experimental import pallas as pl


def kernel(z, embedding):
    raise NotImplementedError("write your pallas kernel here")


# speedup vs baseline: 2.1120x; 2.1120x over previous
import functools

import jax
import jax.numpy as jnp
from jax.experimental import pallas as pl
from jax.experimental.pallas import tpu as pltpu

_B = 16
_C = 64
_HW = 64 * 64
_K = 1024
_T = 512
_NJ = _HW // _T


def _vq_block(z_ref, emb_ref, quant_ref, idx_ref, loss_ref):
    zb = z_ref[0]
    emb = emb_ref[...]
    scores = jax.lax.dot_general(
        emb, zb, (((0,), (0,)), ((), ())),
        preferred_element_type=jnp.float32)
    e_sq = jnp.sum(emb * emb, axis=0)
    z_sq = jnp.sum(zb * zb, axis=0)
    dist = (z_sq[None, :] - 2.0 * scores) + e_sq[:, None]
    idx = jnp.argmin(dist, axis=0)
    onehot = (jax.lax.broadcasted_iota(jnp.int32, (_K, _T), 0)
              == idx[None, :]).astype(jnp.float32)
    quant = jax.lax.dot_general(
        emb, onehot, (((1,), (0,)), ((), ())),
        preferred_element_type=jnp.float32)
    quant_ref[0] = quant
    idx_ref[0, 0] = idx
    diff = quant - zb
    loss_ref[...] = jnp.sum(diff * diff).reshape(1, 1, 1)


@jax.jit
def kernel(z, embedding):
    commitment_cost = 0.25
    z3 = z.reshape(_B, _C, _HW)
    grid = (_B, _NJ)
    quant, idx, loss_parts = pl.pallas_call(
        _vq_block,
        grid=grid,
        in_specs=[
            pl.BlockSpec((1, _C, _T), lambda b, j: (b, 0, j)),
            pl.BlockSpec((_C, _K), lambda b, j: (0, 0)),
        ],
        out_specs=[
            pl.BlockSpec((1, _C, _T), lambda b, j: (b, 0, j)),
            pl.BlockSpec((1, 1, _T), lambda b, j: (b * _NJ + j, 0, 0)),
            pl.BlockSpec((1, 1, 1), lambda b, j: (b * _NJ + j, 0, 0)),
        ],
        out_shape=[
            jax.ShapeDtypeStruct((_B, _C, _HW), jnp.float32),
            jax.ShapeDtypeStruct((_B * _NJ, 1, _T), jnp.int32),
            jax.ShapeDtypeStruct((_B * _NJ, 1, 1), jnp.float32),
        ],
        compiler_params=pltpu.CompilerParams(
            dimension_semantics=("parallel", "parallel")),
    )(z3, embedding)
    quantized_out = quant.reshape(z.shape)
    encoding_indices = idx.reshape(_B, 64, 64)
    loss = (1.0 + commitment_cost) * jnp.sum(loss_parts) / z.size
    return (quantized_out, loss, encoding_indices)


# speedup vs baseline: 2.2630x; 1.0715x over previous
import functools

import jax
import jax.numpy as jnp
from jax.experimental import pallas as pl
from jax.experimental.pallas import tpu as pltpu

_B = 16
_C = 64
_HW = 64 * 64
_K = 1024
_T = 512
_NJ = _HW // _T


def _vq_block(z_ref, emb_ref, quant_ref, idx_ref, loss_ref):
    zb = z_ref[0]
    emb = emb_ref[...]
    scores = jax.lax.dot_general(
        emb, zb, (((0,), (0,)), ((), ())),
        preferred_element_type=jnp.float32)
    h = 0.5 * jnp.sum(emb * emb, axis=0)
    score = scores - h[:, None]
    idx = jnp.argmax(score, axis=0)
    onehot = (jax.lax.broadcasted_iota(jnp.int32, (_K, _T), 0)
              == idx[None, :]).astype(jnp.float32)
    quant = jax.lax.dot_general(
        emb, onehot, (((1,), (0,)), ((), ())),
        preferred_element_type=jnp.float32)
    quant_ref[0] = quant
    idx_ref[0, 0] = idx
    diff = quant - zb
    loss_ref[...] = jnp.sum(diff * diff).reshape(1, 1, 1)


@jax.jit
def kernel(z, embedding):
    commitment_cost = 0.25
    z3 = z.reshape(_B, _C, _HW)
    grid = (_B, _NJ)
    quant, idx, loss_parts = pl.pallas_call(
        _vq_block,
        grid=grid,
        in_specs=[
            pl.BlockSpec((1, _C, _T), lambda b, j: (b, 0, j)),
            pl.BlockSpec((_C, _K), lambda b, j: (0, 0)),
        ],
        out_specs=[
            pl.BlockSpec((1, _C, _T), lambda b, j: (b, 0, j)),
            pl.BlockSpec((1, 1, _T), lambda b, j: (b * _NJ + j, 0, 0)),
            pl.BlockSpec((1, 1, 1), lambda b, j: (b * _NJ + j, 0, 0)),
        ],
        out_shape=[
            jax.ShapeDtypeStruct((_B, _C, _HW), jnp.float32),
            jax.ShapeDtypeStruct((_B * _NJ, 1, _T), jnp.int32),
            jax.ShapeDtypeStruct((_B * _NJ, 1, 1), jnp.float32),
        ],
        compiler_params=pltpu.CompilerParams(
            dimension_semantics=("parallel", "parallel")),
    )(z3, embedding)
    quantized_out = quant.reshape(z.shape)
    encoding_indices = idx.reshape(_B, 64, 64)
    loss = (1.0 + commitment_cost) * jnp.sum(loss_parts) / z.size
    return (quantized_out, loss, encoding_indices)


# speedup vs baseline: 2.8999x; 1.2815x over previous
import functools

import jax
import jax.numpy as jnp
from jax.experimental import pallas as pl
from jax.experimental.pallas import tpu as pltpu

_B = 16
_C = 64
_HW = 64 * 64
_K = 1024
_T = 1024
_NJ = _HW // _T


def _vq_block(z_ref, emb_ref, quant_ref, idx_ref, loss_ref):
    zb = z_ref[0]
    emb = emb_ref[...]
    scores = jax.lax.dot_general(
        emb, zb, (((0,), (0,)), ((), ())),
        preferred_element_type=jnp.float32)
    h = 0.5 * jnp.sum(emb * emb, axis=0)
    score = scores - h[:, None]
    idx = jnp.argmax(score, axis=0)
    onehot = (jax.lax.broadcasted_iota(jnp.int32, (_K, _T), 0)
              == idx[None, :]).astype(jnp.float32)
    quant = jax.lax.dot_general(
        emb, onehot, (((1,), (0,)), ((), ())),
        preferred_element_type=jnp.float32)
    quant_ref[0] = quant
    idx_ref[0, 0] = idx
    diff = quant - zb
    loss_ref[...] = jnp.sum(diff * diff).reshape(1, 1, 1)


@jax.jit
def kernel(z, embedding):
    commitment_cost = 0.25
    z3 = z.reshape(_B, _C, _HW)
    grid = (_B, _NJ)
    quant, idx, loss_parts = pl.pallas_call(
        _vq_block,
        grid=grid,
        in_specs=[
            pl.BlockSpec((1, _C, _T), lambda b, j: (b, 0, j)),
            pl.BlockSpec((_C, _K), lambda b, j: (0, 0)),
        ],
        out_specs=[
            pl.BlockSpec((1, _C, _T), lambda b, j: (b, 0, j)),
            pl.BlockSpec((1, 1, _T), lambda b, j: (b * _NJ + j, 0, 0)),
            pl.BlockSpec((1, 1, 1), lambda b, j: (b * _NJ + j, 0, 0)),
        ],
        out_shape=[
            jax.ShapeDtypeStruct((_B, _C, _HW), jnp.float32),
            jax.ShapeDtypeStruct((_B * _NJ, 1, _T), jnp.int32),
            jax.ShapeDtypeStruct((_B * _NJ, 1, 1), jnp.float32),
        ],
        compiler_params=pltpu.CompilerParams(
            dimension_semantics=("parallel", "parallel")),
    )(z3, embedding)
    quantized_out = quant.reshape(z.shape)
    encoding_indices = idx.reshape(_B, 64, 64)
    loss = (1.0 + commitment_cost) * jnp.sum(loss_parts) / z.size
    return (quantized_out, loss, encoding_indices)


# speedup vs baseline: 3.2296x; 1.1137x over previous
import functools

import jax
import jax.numpy as jnp
from jax.experimental import pallas as pl
from jax.experimental.pallas import tpu as pltpu

_B = 16
_C = 64
_HW = 64 * 64
_K = 1024
_T = 2048
_NJ = _HW // _T


def _vq_block(z_ref, emb_ref, quant_ref, idx_ref, loss_ref):
    zb = z_ref[0]
    emb = emb_ref[...]
    scores = jax.lax.dot_general(
        emb, zb, (((0,), (0,)), ((), ())),
        preferred_element_type=jnp.float32)
    h = 0.5 * jnp.sum(emb * emb, axis=0)
    score = scores - h[:, None]
    idx = jnp.argmax(score, axis=0)
    onehot = (jax.lax.broadcasted_iota(jnp.int32, (_K, _T), 0)
              == idx[None, :]).astype(jnp.float32)
    quant = jax.lax.dot_general(
        emb, onehot, (((1,), (0,)), ((), ())),
        preferred_element_type=jnp.float32)
    quant_ref[0] = quant
    idx_ref[0, 0] = idx
    diff = quant - zb
    loss_ref[...] = jnp.sum(diff * diff).reshape(1, 1, 1)


@jax.jit
def kernel(z, embedding):
    commitment_cost = 0.25
    z3 = z.reshape(_B, _C, _HW)
    grid = (_B, _NJ)
    quant, idx, loss_parts = pl.pallas_call(
        _vq_block,
        grid=grid,
        in_specs=[
            pl.BlockSpec((1, _C, _T), lambda b, j: (b, 0, j)),
            pl.BlockSpec((_C, _K), lambda b, j: (0, 0)),
        ],
        out_specs=[
            pl.BlockSpec((1, _C, _T), lambda b, j: (b, 0, j)),
            pl.BlockSpec((1, 1, _T), lambda b, j: (b * _NJ + j, 0, 0)),
            pl.BlockSpec((1, 1, 1), lambda b, j: (b * _NJ + j, 0, 0)),
        ],
        out_shape=[
            jax.ShapeDtypeStruct((_B, _C, _HW), jnp.float32),
            jax.ShapeDtypeStruct((_B * _NJ, 1, _T), jnp.int32),
            jax.ShapeDtypeStruct((_B * _NJ, 1, 1), jnp.float32),
        ],
        compiler_params=pltpu.CompilerParams(
            dimension_semantics=("parallel", "parallel")),
    )(z3, embedding)
    quantized_out = quant.reshape(z.shape)
    encoding_indices = idx.reshape(_B, 64, 64)
    loss = (1.0 + commitment_cost) * jnp.sum(loss_parts) / z.size
    return (quantized_out, loss, encoding_indices)


# speedup vs baseline: 3.3707x; 1.0437x over previous
import functools

import jax
import jax.numpy as jnp
from jax.experimental import pallas as pl
from jax.experimental.pallas import tpu as pltpu

_B = 16
_C = 64
_HW = 64 * 64
_K = 1024
_T = 4096
_NJ = _HW // _T


def _vq_block(z_ref, emb_ref, quant_ref, idx_ref, loss_ref):
    zb = z_ref[0]
    emb = emb_ref[...]
    scores = jax.lax.dot_general(
        emb, zb, (((0,), (0,)), ((), ())),
        preferred_element_type=jnp.float32)
    h = 0.5 * jnp.sum(emb * emb, axis=0)
    score = scores - h[:, None]
    idx = jnp.argmax(score, axis=0)
    onehot = (jax.lax.broadcasted_iota(jnp.int32, (_K, _T), 0)
              == idx[None, :]).astype(jnp.float32)
    quant = jax.lax.dot_general(
        emb, onehot, (((1,), (0,)), ((), ())),
        preferred_element_type=jnp.float32)
    quant_ref[0] = quant
    idx_ref[0, 0] = idx
    diff = quant - zb
    loss_ref[...] = jnp.sum(diff * diff).reshape(1, 1, 1)


@jax.jit
def kernel(z, embedding):
    commitment_cost = 0.25
    z3 = z.reshape(_B, _C, _HW)
    grid = (_B, _NJ)
    quant, idx, loss_parts = pl.pallas_call(
        _vq_block,
        grid=grid,
        in_specs=[
            pl.BlockSpec((1, _C, _T), lambda b, j: (b, 0, j)),
            pl.BlockSpec((_C, _K), lambda b, j: (0, 0)),
        ],
        out_specs=[
            pl.BlockSpec((1, _C, _T), lambda b, j: (b, 0, j)),
            pl.BlockSpec((1, 1, _T), lambda b, j: (b * _NJ + j, 0, 0)),
            pl.BlockSpec((1, 1, 1), lambda b, j: (b * _NJ + j, 0, 0)),
        ],
        out_shape=[
            jax.ShapeDtypeStruct((_B, _C, _HW), jnp.float32),
            jax.ShapeDtypeStruct((_B * _NJ, 1, _T), jnp.int32),
            jax.ShapeDtypeStruct((_B * _NJ, 1, 1), jnp.float32),
        ],
        compiler_params=pltpu.CompilerParams(
            dimension_semantics=("parallel", "parallel")),
    )(z3, embedding)
    quantized_out = quant.reshape(z.shape)
    encoding_indices = idx.reshape(_B, 64, 64)
    loss = (1.0 + commitment_cost) * jnp.sum(loss_parts) / z.size
    return (quantized_out, loss, encoding_indices)
